# Initial kernel scaffold; baseline (speedup 1.0000x reference)
import jax
import jax.numpy as jnp
from jax.experimental import pallas as pl


def kernel(x, edge_index, W1, b1, W2, b2, Wlin, blin):
    raise NotImplementedError("write your pallas kernel here")


# speedup vs baseline: 15.4557x; 15.4557x over previous
import functools

import jax
import jax.numpy as jnp
from jax import lax
from jax.experimental import pallas as pl
from jax.experimental.pallas import tpu as pltpu
from jax.experimental.pallas import tpu_sc as plsc

N = 10000
D = 128
H = 128
OUT = 2
E = 320000

NC = 2
NS = 16
NW = NC * NS
CHUNK = 128
CPT = -(-E // (NW * CHUNK))
EPAD = NW * CPT * CHUNK
NROWS = 10240
RPT = NROWS // NS
WPT = 624
WTAIL = N - NS * WPT
RB = 400
GRID = N // RB

_mesh = plsc.VectorSubcoreMesh(core_axis_name="c", subcore_axis_name="s")


@functools.partial(
    pl.kernel,
    mesh=_mesh,
    out_type=jax.ShapeDtypeStruct((NC, NROWS), jnp.float32),
    scratch_types=[
        pltpu.VMEM((CHUNK,), jnp.int32),
        pltpu.VMEM((CHUNK,), jnp.float32),
        pltpu.VMEM_SHARED((NROWS,), jnp.float32),
    ],
)
def _deg_kernel(dst_hbm, zeros_hbm, out_hbm, dst_v, ones_v, dacc):
    c = lax.axis_index("c")
    s = lax.axis_index("s")
    wid = c * NS + s
    for i in range(CHUNK // 16):
        ones_v[pl.ds(i * 16, 16)] = jnp.full((16,), 1.0, jnp.float32)
    pltpu.sync_copy(zeros_hbm.at[pl.ds(s * RPT, RPT)], dacc.at[pl.ds(s * RPT, RPT)])
    plsc.subcore_barrier()
    base = wid * CPT

    def body(j, carry):
        pltpu.sync_copy(dst_hbm.at[pl.ds((base + j) * CHUNK, CHUNK)], dst_v)
        pltpu.sync_copy(ones_v, dacc.at[dst_v], add=True)
        return carry

    lax.fori_loop(0, CPT, body, 0)
    plsc.subcore_barrier()
    pltpu.sync_copy(dacc.at[pl.ds(s * RPT, RPT)], out_hbm.at[c, pl.ds(s * RPT, RPT)])


@functools.partial(
    pl.kernel,
    mesh=_mesh,
    out_type=jax.ShapeDtypeStruct((NC, N, H), jnp.float32),
    scratch_types=[
        pltpu.VMEM((CHUNK,), jnp.int32),
        pltpu.VMEM((CHUNK,), jnp.int32),
        pltpu.VMEM((CHUNK, H), jnp.float32),
        pltpu.VMEM_SHARED((NROWS, H), jnp.float32),
        pltpu.SemaphoreType.DMA,
    ],
)
def _scatter_kernel(hs_hbm, src_hbm, dst_hbm, zeros_hbm, out_hbm,
                    src_v, dst_v, rows_v, acc, sem):
    c = lax.axis_index("c")
    s = lax.axis_index("s")
    wid = c * NS + s
    pltpu.sync_copy(zeros_hbm.at[pl.ds(s * RPT, RPT)], acc.at[pl.ds(s * RPT, RPT)])
    plsc.subcore_barrier()
    base = wid * CPT

    def body(j, carry):
        eoff = (base + j) * CHUNK
        pltpu.sync_copy(src_hbm.at[pl.ds(eoff, CHUNK)], src_v)
        pltpu.sync_copy(dst_hbm.at[pl.ds(eoff, CHUNK)], dst_v)
        pltpu.async_copy(hs_hbm.at[src_v], rows_v, sem).wait()
        pltpu.sync_copy(rows_v, acc.at[dst_v], add=True)
        return carry

    lax.fori_loop(0, CPT, body, 0)
    plsc.subcore_barrier()
    pltpu.sync_copy(acc.at[pl.ds(s * WPT, WPT)], out_hbm.at[c, pl.ds(s * WPT, WPT)])

    @pl.when(s == NS - 1)
    def _tail():
        pltpu.sync_copy(acc.at[pl.ds(NS * WPT, WTAIL)],
                        out_hbm.at[c, pl.ds(NS * WPT, WTAIL)])


def _dinv_body(degp_ref, o_ref):
    o_ref[...] = lax.rsqrt(degp_ref[0] + degp_ref[1] + 1.0)


def _dinv_call(degp):
    return pl.pallas_call(
        _dinv_body,
        out_shape=jax.ShapeDtypeStruct((NROWS // 128, 128), jnp.float32),
    )(degp)


def _mm1_body(x_ref, w_ref, dinv_ref, o_ref):
    o_ref[...] = jnp.dot(x_ref[...], w_ref[...],
                         preferred_element_type=jnp.float32) * dinv_ref[...]


def _mm1_call(x, W, dinv_col):
    return pl.pallas_call(
        _mm1_body,
        grid=(GRID,),
        in_specs=[
            pl.BlockSpec((RB, D), lambda i: (i, 0)),
            pl.BlockSpec((D, H), lambda i: (0, 0)),
            pl.BlockSpec((RB, 1), lambda i: (i, 0)),
        ],
        out_specs=pl.BlockSpec((RB, H), lambda i: (i, 0)),
        out_shape=jax.ShapeDtypeStruct((N, H), jnp.float32),
    )(x, W, dinv_col)


def _mm2_body(agg_ref, hs_ref, dinv_ref, b_ref, w_ref, o_ref):
    a = (agg_ref[0] + agg_ref[1] + hs_ref[...]) * dinv_ref[...] + b_ref[...]
    t = jnp.maximum(a, 0.0)
    o_ref[...] = jnp.dot(t, w_ref[...],
                         preferred_element_type=jnp.float32) * dinv_ref[...]


def _mm2_call(agg, hs, dinv_col, b_row, W):
    return pl.pallas_call(
        _mm2_body,
        grid=(GRID,),
        in_specs=[
            pl.BlockSpec((NC, RB, H), lambda i: (0, i, 0)),
            pl.BlockSpec((RB, H), lambda i: (i, 0)),
            pl.BlockSpec((RB, 1), lambda i: (i, 0)),
            pl.BlockSpec((1, H), lambda i: (0, 0)),
            pl.BlockSpec((H, H), lambda i: (0, 0)),
        ],
        out_specs=pl.BlockSpec((RB, H), lambda i: (i, 0)),
        out_shape=jax.ShapeDtypeStruct((N, H), jnp.float32),
    )(agg, hs, dinv_col, b_row, W)


def _mm3_body(agg_ref, hs_ref, dinv_ref, b_ref, w_ref, blin_ref, o_ref):
    a = (agg_ref[0] + agg_ref[1] + hs_ref[...]) * dinv_ref[...] + b_ref[...]
    t = jnp.maximum(a, 0.0)
    o_ref[...] = jnp.dot(t, w_ref[...],
                         preferred_element_type=jnp.float32) + blin_ref[...]


def _mm3_call(agg, hs, dinv_col, b_row, Wp, blin_row):
    return pl.pallas_call(
        _mm3_body,
        grid=(GRID,),
        in_specs=[
            pl.BlockSpec((NC, RB, H), lambda i: (0, i, 0)),
            pl.BlockSpec((RB, H), lambda i: (i, 0)),
            pl.BlockSpec((RB, 1), lambda i: (i, 0)),
            pl.BlockSpec((1, H), lambda i: (0, 0)),
            pl.BlockSpec((H, 128), lambda i: (0, 0)),
            pl.BlockSpec((1, 128), lambda i: (0, 0)),
        ],
        out_specs=pl.BlockSpec((RB, 128), lambda i: (i, 0)),
        out_shape=jax.ShapeDtypeStruct((N, 128), jnp.float32),
    )(agg, hs, dinv_col, b_row, Wp, blin_row)


def kernel(x, edge_index, W1, b1, W2, b2, Wlin, blin):
    src = edge_index[0]
    dst = edge_index[1]
    pad = EPAD - E
    ar = jnp.arange(pad, dtype=jnp.int32)
    pad_src = (ar * 7919) % N
    pad_dst = N + ar % (NROWS - N)
    src_p = jnp.concatenate([src, pad_src])
    dst_p = jnp.concatenate([dst, pad_dst])

    zeros1 = jnp.zeros((NROWS,), jnp.float32)
    zeros2 = jnp.zeros((NROWS, H), jnp.float32)

    degp = _deg_kernel(dst_p, zeros1)
    dinv2d = _dinv_call(degp.reshape(NC, NROWS // 128, 128))
    dinv_col = dinv2d.reshape(NROWS, 1)[:N]

    hs1 = _mm1_call(x, W1, dinv_col)
    agg1 = _scatter_kernel(hs1, src_p, dst_p, zeros2)
    hs2 = _mm2_call(agg1, hs1, dinv_col, b1.reshape(1, H), W2)
    agg2 = _scatter_kernel(hs2, src_p, dst_p, zeros2)

    Wp = jnp.zeros((H, 128), jnp.float32).at[:, :OUT].set(Wlin)
    blin_row = jnp.zeros((1, 128), jnp.float32).at[0, :OUT].set(blin)
    out = _mm3_call(agg2, hs2, dinv_col, b2.reshape(1, H), Wp, blin_row)
    return out[:, :OUT]


# speedup vs baseline: 29.5326x; 1.9108x over previous
import functools

import jax
import jax.numpy as jnp
from jax import lax
from jax.experimental import pallas as pl
from jax.experimental.pallas import tpu as pltpu
from jax.experimental.pallas import tpu_sc as plsc

N = 10000
D = 128
H = 128
OUT = 2
E = 320000

NC = 2
NS = 16
NW = NC * NS
CHUNK = 128
CPT = 80
HCPT = CPT // 2
EPAD = NW * CPT * CHUNK
NBUF = 2
NROWS = 10112
RPT = NROWS // NS
WPT = 624
WTAIL = N - NS * WPT
RB = 400
GRID = N // RB

_mesh = plsc.VectorSubcoreMesh(core_axis_name="c", subcore_axis_name="s")


@functools.partial(
    pl.kernel,
    mesh=_mesh,
    out_type=jax.ShapeDtypeStruct((NC * NROWS,), jnp.float32),
    scratch_types=[
        pltpu.VMEM((CPT, CHUNK), jnp.int32),
        pltpu.VMEM((CHUNK,), jnp.float32),
        pltpu.VMEM_SHARED((NROWS,), jnp.float32),
        pltpu.SemaphoreType.DMA,
    ],
)
def _deg_kernel(dst_hbm, zeros_hbm, out_hbm, dst_all, ones_v, dacc, dsem):
    c = lax.axis_index("c")
    s = lax.axis_index("s")
    wid = c * NS + s
    for i in range(CHUNK // 16):
        ones_v[pl.ds(i * 16, 16)] = jnp.full((16,), 1.0, jnp.float32)
    pltpu.sync_copy(dst_hbm.at[wid], dst_all)

    @pl.when(s < NS - 1)
    def _z0():
        pltpu.sync_copy(zeros_hbm.at[pl.ds(s * 640, 640)],
                        dacc.at[pl.ds(s * 640, 640)])

    @pl.when(s == NS - 1)
    def _z1():
        pltpu.sync_copy(zeros_hbm.at[pl.ds((NS - 1) * 640, NROWS - (NS - 1) * 640)],
                        dacc.at[pl.ds((NS - 1) * 640, NROWS - (NS - 1) * 640)])

    plsc.subcore_barrier()

    def body(g, carry):
        for i in range(8):
            pltpu.async_copy(ones_v, dacc.at[dst_all.at[g * 8 + i]], dsem,
                             add=True)
        for i in range(8):
            pltpu.make_async_copy(ones_v, dacc.at[dst_all.at[g * 8 + i]],
                                  dsem).wait()
        return carry

    lax.fori_loop(0, CPT // 8, body, 0)
    plsc.subcore_barrier()

    @pl.when(s < NS - 1)
    def _w0():
        pltpu.sync_copy(dacc.at[pl.ds(s * 640, 640)],
                        out_hbm.at[pl.ds(c * NROWS + s * 640, 640)])

    @pl.when(s == NS - 1)
    def _w1():
        pltpu.sync_copy(dacc.at[pl.ds((NS - 1) * 640, NROWS - (NS - 1) * 640)],
                        out_hbm.at[pl.ds(c * NROWS + (NS - 1) * 640,
                                         NROWS - (NS - 1) * 640)])


@functools.partial(
    pl.kernel,
    mesh=_mesh,
    out_type=jax.ShapeDtypeStruct((NC, N, H), jnp.float32),
    scratch_types=[
        pltpu.VMEM((HCPT, CHUNK), jnp.int32),
        pltpu.VMEM((HCPT, CHUNK), jnp.int32),
        pltpu.VMEM_SHARED((NROWS, H), jnp.float32),
    ]
    + [pltpu.VMEM((CHUNK, H), jnp.float32) for _ in range(NBUF)]
    + [pltpu.SemaphoreType.DMA for _ in range(2 * NBUF)],
)
def _scatter_kernel(hs_hbm, src_hbm, dst_hbm, zeros_hbm, out_hbm,
                    src_half, dst_half, acc, *bufs_sems):
    bufs = bufs_sems[:NBUF]
    gsem = bufs_sems[NBUF:2 * NBUF]
    ssem = bufs_sems[2 * NBUF:]
    c = lax.axis_index("c")
    s = lax.axis_index("s")
    wid = c * NS + s
    pltpu.sync_copy(zeros_hbm.at[pl.ds(s * RPT, RPT)], acc.at[pl.ds(s * RPT, RPT)])
    plsc.subcore_barrier()

    def gather(j, b):
        pltpu.async_copy(hs_hbm.at[src_half.at[j]], bufs[b], gsem[b])

    def wait_gather(j, b):
        pltpu.make_async_copy(hs_hbm.at[src_half.at[j]], bufs[b], gsem[b]).wait()

    def scat(j, b):
        pltpu.async_copy(bufs[b], acc.at[dst_half.at[j]], ssem[b], add=True)

    def wait_scat(j, b):
        pltpu.make_async_copy(bufs[b], acc.at[dst_half.at[j]], ssem[b]).wait()

    for h in range(2):
        pltpu.sync_copy(src_hbm.at[wid, pl.ds(h * HCPT, HCPT)], src_half)
        pltpu.sync_copy(dst_hbm.at[wid, pl.ds(h * HCPT, HCPT)], dst_half)
        gather(0, 0)
        gather(1, 1)

        def body(g, carry):
            for b in range(NBUF):
                j = g * NBUF + b
                wait_gather(j, b)
                scat(j, b)

                @pl.when(j + NBUF < HCPT)
                def _refill():
                    wait_scat(j, b)
                    gather(j + NBUF, b)
            return carry

        lax.fori_loop(0, HCPT // NBUF, body, 0)
        for b in range(NBUF):
            wait_scat(HCPT - NBUF + b, b)
    plsc.subcore_barrier()
    pltpu.sync_copy(acc.at[pl.ds(s * WPT, WPT)],
                    out_hbm.at[c, pl.ds(s * WPT, WPT)])

    @pl.when(s == NS - 1)
    def _tail():
        pltpu.sync_copy(acc.at[pl.ds(NS * WPT, WTAIL)],
                        out_hbm.at[c, pl.ds(NS * WPT, WTAIL)])


def _dinv_body(degp_ref, o_ref):
    o_ref[...] = lax.rsqrt(degp_ref[0] + degp_ref[1] + 1.0)


def _dinv_call(degp):
    return pl.pallas_call(
        _dinv_body,
        out_shape=jax.ShapeDtypeStruct((NROWS // 128, 128), jnp.float32),
    )(degp)


def _mm1_body(x_ref, w_ref, dinv_ref, o_ref):
    o_ref[...] = jnp.dot(x_ref[...], w_ref[...],
                         preferred_element_type=jnp.float32) * dinv_ref[...]


def _mm1_call(x, W, dinv_col):
    return pl.pallas_call(
        _mm1_body,
        grid=(GRID,),
        in_specs=[
            pl.BlockSpec((RB, D), lambda i: (i, 0)),
            pl.BlockSpec((D, H), lambda i: (0, 0)),
            pl.BlockSpec((RB, 1), lambda i: (i, 0)),
        ],
        out_specs=pl.BlockSpec((RB, H), lambda i: (i, 0)),
        out_shape=jax.ShapeDtypeStruct((N, H), jnp.float32),
    )(x, W, dinv_col)


def _mm2_body(agg_ref, hs_ref, dinv_ref, b_ref, w_ref, o_ref):
    a = (agg_ref[0] + agg_ref[1] + hs_ref[...]) * dinv_ref[...] + b_ref[...]
    t = jnp.maximum(a, 0.0)
    o_ref[...] = jnp.dot(t, w_ref[...],
                         preferred_element_type=jnp.float32) * dinv_ref[...]


def _mm2_call(agg, hs, dinv_col, b_row, W):
    return pl.pallas_call(
        _mm2_body,
        grid=(GRID,),
        in_specs=[
            pl.BlockSpec((NC, RB, H), lambda i: (0, i, 0)),
            pl.BlockSpec((RB, H), lambda i: (i, 0)),
            pl.BlockSpec((RB, 1), lambda i: (i, 0)),
            pl.BlockSpec((1, H), lambda i: (0, 0)),
            pl.BlockSpec((H, H), lambda i: (0, 0)),
        ],
        out_specs=pl.BlockSpec((RB, H), lambda i: (i, 0)),
        out_shape=jax.ShapeDtypeStruct((N, H), jnp.float32),
    )(agg, hs, dinv_col, b_row, W)


def _mm3_body(agg_ref, hs_ref, dinv_ref, b_ref, w_ref, blin_ref, o_ref):
    a = (agg_ref[0] + agg_ref[1] + hs_ref[...]) * dinv_ref[...] + b_ref[...]
    t = jnp.maximum(a, 0.0)
    o_ref[...] = jnp.dot(t, w_ref[...],
                         preferred_element_type=jnp.float32) + blin_ref[...]


def _mm3_call(agg, hs, dinv_col, b_row, Wp, blin_row):
    return pl.pallas_call(
        _mm3_body,
        grid=(GRID,),
        in_specs=[
            pl.BlockSpec((NC, RB, H), lambda i: (0, i, 0)),
            pl.BlockSpec((RB, H), lambda i: (i, 0)),
            pl.BlockSpec((RB, 1), lambda i: (i, 0)),
            pl.BlockSpec((1, H), lambda i: (0, 0)),
            pl.BlockSpec((H, 128), lambda i: (0, 0)),
            pl.BlockSpec((1, 128), lambda i: (0, 0)),
        ],
        out_specs=pl.BlockSpec((RB, 128), lambda i: (i, 0)),
        out_shape=jax.ShapeDtypeStruct((N, 128), jnp.float32),
    )(agg, hs, dinv_col, b_row, Wp, blin_row)


def kernel(x, edge_index, W1, b1, W2, b2, Wlin, blin):
    src = edge_index[0]
    dst = edge_index[1]
    pad = EPAD - E
    ar = jnp.arange(pad, dtype=jnp.int32)
    pad_src = (ar * 7919) % N
    pad_dst = N + ar % (NROWS - N)
    src_p = jnp.concatenate([src, pad_src]).reshape(NW, CPT, CHUNK)
    dst_p = jnp.concatenate([dst, pad_dst]).reshape(NW, CPT, CHUNK)

    zeros1 = jnp.zeros((NROWS,), jnp.float32)
    zeros2 = jnp.zeros((NROWS, H), jnp.float32)

    degp = _deg_kernel(dst_p, zeros1)
    dinv2d = _dinv_call(degp.reshape(NC, NROWS // 128, 128))
    dinv_col = dinv2d.reshape(NROWS, 1)[:N]

    hs1 = _mm1_call(x, W1, dinv_col)
    agg1 = _scatter_kernel(hs1, src_p, dst_p, zeros2)
    hs2 = _mm2_call(agg1, hs1, dinv_col, b1.reshape(1, H), W2)
    agg2 = _scatter_kernel(hs2, src_p, dst_p, zeros2)

    Wp = jnp.zeros((H, 128), jnp.float32).at[:, :OUT].set(Wlin)
    blin_row = jnp.zeros((1, 128), jnp.float32).at[0, :OUT].set(blin)
    out = _mm3_call(agg2, hs2, dinv_col, b2.reshape(1, H), Wp, blin_row)
    return out[:, :OUT]


# speedup vs baseline: 31.5207x; 1.0673x over previous
import functools

import jax
import jax.numpy as jnp
import numpy as np
from jax import lax
from jax.experimental import pallas as pl
from jax.experimental.pallas import tpu as pltpu
from jax.experimental.pallas import tpu_sc as plsc

N = 10000
D = 128
H = 128
OUT = 2
E = 320000

NC = 2
NS = 16
NW = NC * NS
CHUNK = 64
CPT = 160
HCPT = CPT // 2
EPAD = NW * CPT * CHUNK
NBUF = 3
NROWS = 10112
RPT = NROWS // NS
WPT = 624
WTAIL = N - NS * WPT
RB = 2000
GRID = N // RB

_PAD = EPAD - E
_PAD_SRC = np.asarray((np.arange(_PAD) * 7919) % N, np.int32)
_PAD_DST = np.asarray(N + np.arange(_PAD) % (NROWS - N), np.int32)
_ZEROS1 = np.zeros((NROWS,), np.float32)
_ZEROS2 = np.zeros((NROWS, H), np.float32)

_mesh = plsc.VectorSubcoreMesh(core_axis_name="c", subcore_axis_name="s")


@functools.partial(
    pl.kernel,
    mesh=_mesh,
    out_type=jax.ShapeDtypeStruct((NC * NROWS,), jnp.float32),
    scratch_types=[
        pltpu.VMEM((CPT, CHUNK), jnp.int32),
        pltpu.VMEM((CHUNK,), jnp.float32),
        pltpu.VMEM_SHARED((NROWS,), jnp.float32),
        pltpu.SemaphoreType.DMA,
    ],
)
def _deg_kernel(dst_hbm, zeros_hbm, out_hbm, dst_all, ones_v, dacc, dsem):
    c = lax.axis_index("c")
    s = lax.axis_index("s")
    wid = c * NS + s
    for i in range(CHUNK // 16):
        ones_v[pl.ds(i * 16, 16)] = jnp.full((16,), 1.0, jnp.float32)
    pltpu.sync_copy(dst_hbm.at[wid], dst_all)

    @pl.when(s < NS - 1)
    def _z0():
        pltpu.sync_copy(zeros_hbm.at[pl.ds(s * 640, 640)],
                        dacc.at[pl.ds(s * 640, 640)])

    @pl.when(s == NS - 1)
    def _z1():
        pltpu.sync_copy(zeros_hbm.at[pl.ds((NS - 1) * 640, NROWS - (NS - 1) * 640)],
                        dacc.at[pl.ds((NS - 1) * 640, NROWS - (NS - 1) * 640)])

    plsc.subcore_barrier()

    def body(g, carry):
        for i in range(8):
            pltpu.async_copy(ones_v, dacc.at[dst_all.at[g * 8 + i]], dsem,
                             add=True)
        for i in range(8):
            pltpu.make_async_copy(ones_v, dacc.at[dst_all.at[g * 8 + i]],
                                  dsem).wait()
        return carry

    lax.fori_loop(0, CPT // 8, body, 0)
    plsc.subcore_barrier()

    @pl.when(s < NS - 1)
    def _w0():
        pltpu.sync_copy(dacc.at[pl.ds(s * 640, 640)],
                        out_hbm.at[pl.ds(c * NROWS + s * 640, 640)])

    @pl.when(s == NS - 1)
    def _w1():
        pltpu.sync_copy(dacc.at[pl.ds((NS - 1) * 640, NROWS - (NS - 1) * 640)],
                        out_hbm.at[pl.ds(c * NROWS + (NS - 1) * 640,
                                         NROWS - (NS - 1) * 640)])


@functools.partial(
    pl.kernel,
    mesh=_mesh,
    out_type=jax.ShapeDtypeStruct((NC, N, H), jnp.float32),
    scratch_types=[
        pltpu.VMEM((HCPT, CHUNK), jnp.int32),
        pltpu.VMEM((HCPT, CHUNK), jnp.int32),
        pltpu.VMEM_SHARED((NROWS, H), jnp.float32),
    ]
    + [pltpu.VMEM((CHUNK, H), jnp.float32) for _ in range(NBUF)]
    + [pltpu.SemaphoreType.DMA for _ in range(2 * NBUF)],
)
def _scatter_kernel(hs_hbm, src_hbm, dst_hbm, zeros_hbm, out_hbm,
                    src_half, dst_half, acc, *bufs_sems):
    bufs = bufs_sems[:NBUF]
    gsem = bufs_sems[NBUF:2 * NBUF]
    ssem = bufs_sems[2 * NBUF:]
    c = lax.axis_index("c")
    s = lax.axis_index("s")
    wid = c * NS + s
    pltpu.sync_copy(zeros_hbm.at[pl.ds(s * RPT, RPT)], acc.at[pl.ds(s * RPT, RPT)])
    plsc.subcore_barrier()

    def gather(j, b):
        pltpu.async_copy(hs_hbm.at[src_half.at[j]], bufs[b], gsem[b])

    def wait_gather(j, b):
        pltpu.make_async_copy(hs_hbm.at[src_half.at[j]], bufs[b], gsem[b]).wait()

    def scat(j, b):
        pltpu.async_copy(bufs[b], acc.at[dst_half.at[j]], ssem[b], add=True)

    def wait_scat(j, b):
        pltpu.make_async_copy(bufs[b], acc.at[dst_half.at[j]], ssem[b]).wait()

    for h in range(2):
        pltpu.sync_copy(src_hbm.at[wid, pl.ds(h * HCPT, HCPT)], src_half)
        pltpu.sync_copy(dst_hbm.at[wid, pl.ds(h * HCPT, HCPT)], dst_half)
        gather(0, 0)
        gather(1, 1)

        def body(g, carry):
            for b in range(NBUF):
                j = g * NBUF + b
                b2 = (b + 2) % NBUF
                wait_gather(j, b)
                scat(j, b)

                @pl.when(j >= 1)
                def _drain():
                    wait_scat(j - 1, b2)

                @pl.when(j + 2 < HCPT)
                def _refill():
                    gather(j + 2, b2)
            return carry

        lax.fori_loop(0, HCPT // NBUF, body, 0)
        for j in range(NBUF * (HCPT // NBUF), HCPT):
            b = j % NBUF
            wait_gather(j, b)
            scat(j, b)
            wait_scat(j - 1, (b + 2) % NBUF)
        wait_scat(HCPT - 1, (HCPT - 1) % NBUF)
    plsc.subcore_barrier()
    pltpu.sync_copy(acc.at[pl.ds(s * WPT, WPT)],
                    out_hbm.at[c, pl.ds(s * WPT, WPT)])

    @pl.when(s == NS - 1)
    def _tail():
        pltpu.sync_copy(acc.at[pl.ds(NS * WPT, WTAIL)],
                        out_hbm.at[c, pl.ds(NS * WPT, WTAIL)])


def _deg_spec():
    return pl.BlockSpec((NC, RB, 1), lambda i: (0, i, 0))


def _dinv_of(deg_ref):
    return lax.rsqrt(deg_ref[0] + deg_ref[1] + 1.0)


def _mm1_body(x_ref, w_ref, deg_ref, o_ref):
    o_ref[...] = jnp.dot(x_ref[...], w_ref[...],
                         preferred_element_type=jnp.float32) * _dinv_of(deg_ref)


def _mm1_call(x, W, degc):
    return pl.pallas_call(
        _mm1_body,
        grid=(GRID,),
        in_specs=[
            pl.BlockSpec((RB, D), lambda i: (i, 0)),
            pl.BlockSpec((D, H), lambda i: (0, 0)),
            _deg_spec(),
        ],
        out_specs=pl.BlockSpec((RB, H), lambda i: (i, 0)),
        out_shape=jax.ShapeDtypeStruct((N, H), jnp.float32),
    )(x, W, degc)


def _mm2_body(agg_ref, hs_ref, deg_ref, b_ref, w_ref, o_ref):
    dinv = _dinv_of(deg_ref)
    a = (agg_ref[0] + agg_ref[1] + hs_ref[...]) * dinv + b_ref[...]
    t = jnp.maximum(a, 0.0)
    o_ref[...] = jnp.dot(t, w_ref[...],
                         preferred_element_type=jnp.float32) * dinv


def _mm2_call(agg, hs, degc, b_row, W):
    return pl.pallas_call(
        _mm2_body,
        grid=(GRID,),
        in_specs=[
            pl.BlockSpec((NC, RB, H), lambda i: (0, i, 0)),
            pl.BlockSpec((RB, H), lambda i: (i, 0)),
            _deg_spec(),
            pl.BlockSpec((1, H), lambda i: (0, 0)),
            pl.BlockSpec((H, H), lambda i: (0, 0)),
        ],
        out_specs=pl.BlockSpec((RB, H), lambda i: (i, 0)),
        out_shape=jax.ShapeDtypeStruct((N, H), jnp.float32),
    )(agg, hs, degc, b_row, W)


def _mm3_body(agg_ref, hs_ref, deg_ref, b_ref, w_ref, blin_ref, o_ref):
    dinv = _dinv_of(deg_ref)
    a = (agg_ref[0] + agg_ref[1] + hs_ref[...]) * dinv + b_ref[...]
    t = jnp.maximum(a, 0.0)
    o_ref[...] = jnp.dot(t, w_ref[...],
                         preferred_element_type=jnp.float32) + blin_ref[...]


def _mm3_call(agg, hs, degc, b_row, Wlin, blin_row):
    return pl.pallas_call(
        _mm3_body,
        grid=(GRID,),
        in_specs=[
            pl.BlockSpec((NC, RB, H), lambda i: (0, i, 0)),
            pl.BlockSpec((RB, H), lambda i: (i, 0)),
            _deg_spec(),
            pl.BlockSpec((1, H), lambda i: (0, 0)),
            pl.BlockSpec((H, OUT), lambda i: (0, 0)),
            pl.BlockSpec((1, OUT), lambda i: (0, 0)),
        ],
        out_specs=pl.BlockSpec((RB, OUT), lambda i: (i, 0)),
        out_shape=jax.ShapeDtypeStruct((N, OUT), jnp.float32),
    )(agg, hs, degc, b_row, Wlin, blin_row)


def kernel(x, edge_index, W1, b1, W2, b2, Wlin, blin):
    src = edge_index[0]
    dst = edge_index[1]
    src_p = jnp.concatenate([src, jnp.asarray(_PAD_SRC)]).reshape(NW, CPT, CHUNK)
    dst_p = jnp.concatenate([dst, jnp.asarray(_PAD_DST)]).reshape(NW, CPT, CHUNK)
    zeros1 = jnp.asarray(_ZEROS1)
    zeros2 = jnp.asarray(_ZEROS2)

    degp = _deg_kernel(dst_p, zeros1)
    degc = degp.reshape(NC, NROWS, 1)

    hs1 = _mm1_call(x, W1, degc)
    agg1 = _scatter_kernel(hs1, src_p, dst_p, zeros2)
    hs2 = _mm2_call(agg1, hs1, degc, b1.reshape(1, H), W2)
    agg2 = _scatter_kernel(hs2, src_p, dst_p, zeros2)
    return _mm3_call(agg2, hs2, degc, b2.reshape(1, H), Wlin,
                     blin.reshape(1, OUT))


# speedup vs baseline: 32.7954x; 1.0404x over previous
import functools

import jax
import jax.numpy as jnp
import numpy as np
from jax import lax
from jax.experimental import pallas as pl
from jax.experimental.pallas import tpu as pltpu
from jax.experimental.pallas import tpu_sc as plsc

N = 10000
D = 128
H = 128
OUT = 2
E = 320000

NC = 2
NS = 16
NW = NC * NS
CHUNK = 64
CPT = 160
HCPT = CPT // 2
EPAD = NW * CPT * CHUNK
NBUF = 3
NROWS = 10112
RPT = NROWS // NS
WPT = 624
WTAIL = N - NS * WPT
RB = 2000
GRID = N // RB

_PAD = EPAD - E
_PAD_SRC = np.asarray((np.arange(_PAD) * 7919) % N, np.int32)
_PAD_DST = np.asarray(N + np.arange(_PAD) % (NROWS - N), np.int32)
_ZEROS1 = np.zeros((NROWS,), np.float32)
_ZEROS2 = np.zeros((NROWS, H), np.float32)

_mesh = plsc.VectorSubcoreMesh(core_axis_name="c", subcore_axis_name="s")


@functools.partial(
    pl.kernel,
    mesh=_mesh,
    out_type=jax.ShapeDtypeStruct((NC * NROWS,), jnp.float32),
    scratch_types=[
        pltpu.VMEM((CPT, CHUNK), jnp.int32),
        pltpu.VMEM((CHUNK,), jnp.float32),
        pltpu.VMEM_SHARED((NROWS,), jnp.float32),
        pltpu.SemaphoreType.DMA,
    ],
)
def _deg_kernel(dst_hbm, zeros_hbm, out_hbm, dst_all, ones_v, dacc, dsem):
    c = lax.axis_index("c")
    s = lax.axis_index("s")
    wid = c * NS + s
    for i in range(CHUNK // 16):
        ones_v[pl.ds(i * 16, 16)] = jnp.full((16,), 1.0, jnp.float32)
    pltpu.sync_copy(dst_hbm.at[wid], dst_all)

    @pl.when(s < NS - 1)
    def _z0():
        pltpu.sync_copy(zeros_hbm.at[pl.ds(s * 640, 640)],
                        dacc.at[pl.ds(s * 640, 640)])

    @pl.when(s == NS - 1)
    def _z1():
        pltpu.sync_copy(zeros_hbm.at[pl.ds((NS - 1) * 640, NROWS - (NS - 1) * 640)],
                        dacc.at[pl.ds((NS - 1) * 640, NROWS - (NS - 1) * 640)])

    plsc.subcore_barrier()

    def body(g, carry):
        for i in range(8):
            pltpu.async_copy(ones_v, dacc.at[dst_all.at[g * 8 + i]], dsem,
                             add=True)
        for i in range(8):
            pltpu.make_async_copy(ones_v, dacc.at[dst_all.at[g * 8 + i]],
                                  dsem).wait()
        return carry

    lax.fori_loop(0, CPT // 8, body, 0)
    plsc.subcore_barrier()

    @pl.when(s < NS - 1)
    def _w0():
        pltpu.sync_copy(dacc.at[pl.ds(s * 640, 640)],
                        out_hbm.at[pl.ds(c * NROWS + s * 640, 640)])

    @pl.when(s == NS - 1)
    def _w1():
        pltpu.sync_copy(dacc.at[pl.ds((NS - 1) * 640, NROWS - (NS - 1) * 640)],
                        out_hbm.at[pl.ds(c * NROWS + (NS - 1) * 640,
                                         NROWS - (NS - 1) * 640)])


@functools.partial(
    pl.kernel,
    mesh=_mesh,
    out_type=jax.ShapeDtypeStruct((NC, N, H), jnp.float32),
    scratch_types=[
        pltpu.VMEM((HCPT, CHUNK), jnp.int32),
        pltpu.VMEM((HCPT, CHUNK), jnp.int32),
        pltpu.VMEM_SHARED((NROWS, H), jnp.float32),
    ]
    + [pltpu.VMEM((CHUNK, H), jnp.float32) for _ in range(NBUF)]
    + [pltpu.SemaphoreType.DMA for _ in range(2 * NBUF)],
)
def _scatter_kernel(hs_hbm, src_hbm, dst_hbm, out_hbm,
                    src_half, dst_half, acc, *bufs_sems):
    bufs = bufs_sems[:NBUF]
    gsem = bufs_sems[NBUF:2 * NBUF]
    ssem = bufs_sems[2 * NBUF:]
    c = lax.axis_index("c")
    s = lax.axis_index("s")
    wid = c * NS + s

    def zrow(j, carry):
        for i in range(H // 16):
            bufs[0][j, pl.ds(i * 16, 16)] = jnp.zeros((16,), jnp.float32)
        return carry

    lax.fori_loop(0, CHUNK, zrow, 0)
    nfull = RPT // CHUNK
    for k in range(nfull):
        pltpu.sync_copy(bufs[0], acc.at[pl.ds(s * RPT + k * CHUNK, CHUNK)])
    if RPT % CHUNK:
        pltpu.sync_copy(bufs[0].at[pl.ds(0, RPT % CHUNK)],
                        acc.at[pl.ds(s * RPT + nfull * CHUNK, RPT % CHUNK)])
    plsc.subcore_barrier()

    def gather(j, b):
        pltpu.async_copy(hs_hbm.at[src_half.at[j]], bufs[b], gsem[b])

    def wait_gather(j, b):
        pltpu.make_async_copy(hs_hbm.at[src_half.at[j]], bufs[b], gsem[b]).wait()

    def scat(j, b):
        pltpu.async_copy(bufs[b], acc.at[dst_half.at[j]], ssem[b], add=True)

    def wait_scat(j, b):
        pltpu.make_async_copy(bufs[b], acc.at[dst_half.at[j]], ssem[b]).wait()

    for h in range(2):
        pltpu.sync_copy(src_hbm.at[wid, pl.ds(h * HCPT, HCPT)], src_half)
        pltpu.sync_copy(dst_hbm.at[wid, pl.ds(h * HCPT, HCPT)], dst_half)
        gather(0, 0)
        gather(1, 1)

        def body(g, carry):
            for b in range(NBUF):
                j = g * NBUF + b
                b2 = (b + 2) % NBUF
                wait_gather(j, b)
                scat(j, b)

                @pl.when(j >= 1)
                def _drain():
                    wait_scat(j - 1, b2)

                @pl.when(j + 2 < HCPT)
                def _refill():
                    gather(j + 2, b2)
            return carry

        lax.fori_loop(0, HCPT // NBUF, body, 0)
        for j in range(NBUF * (HCPT // NBUF), HCPT):
            b = j % NBUF
            wait_gather(j, b)
            scat(j, b)
            wait_scat(j - 1, (b + 2) % NBUF)
        wait_scat(HCPT - 1, (HCPT - 1) % NBUF)
    plsc.subcore_barrier()
    pltpu.sync_copy(acc.at[pl.ds(s * WPT, WPT)],
                    out_hbm.at[c, pl.ds(s * WPT, WPT)])

    @pl.when(s == NS - 1)
    def _tail():
        pltpu.sync_copy(acc.at[pl.ds(NS * WPT, WTAIL)],
                        out_hbm.at[c, pl.ds(NS * WPT, WTAIL)])


def _dinv_body(degp_ref, o_ref):
    o_ref[...] = lax.rsqrt(degp_ref[0] + degp_ref[1] + 1.0)


def _dinv_call(degp):
    return pl.pallas_call(
        _dinv_body,
        out_shape=jax.ShapeDtypeStruct((NROWS // 128, 128), jnp.float32),
    )(degp)


def _dinv_spec():
    return pl.BlockSpec((RB, H), lambda i: (i, 0))


def _mm1_body(x_ref, w_ref, dinv_ref, o_ref):
    o_ref[...] = jnp.dot(x_ref[...], w_ref[...],
                         preferred_element_type=jnp.float32) * dinv_ref[...]


def _mm1_call(x, W, dinv_rep):
    return pl.pallas_call(
        _mm1_body,
        grid=(GRID,),
        in_specs=[
            pl.BlockSpec((RB, D), lambda i: (i, 0)),
            pl.BlockSpec((D, H), lambda i: (0, 0)),
            _dinv_spec(),
        ],
        out_specs=pl.BlockSpec((RB, H), lambda i: (i, 0)),
        out_shape=jax.ShapeDtypeStruct((N, H), jnp.float32),
    )(x, W, dinv_rep)


def _mm2_body(agg_ref, hs_ref, dinv_ref, b_ref, w_ref, o_ref):
    dinv = dinv_ref[...]
    a = (agg_ref[0] + agg_ref[1] + hs_ref[...]) * dinv + b_ref[...]
    t = jnp.maximum(a, 0.0)
    o_ref[...] = jnp.dot(t, w_ref[...],
                         preferred_element_type=jnp.float32) * dinv


def _mm2_call(agg, hs, dinv_rep, b_row, W):
    return pl.pallas_call(
        _mm2_body,
        grid=(GRID,),
        in_specs=[
            pl.BlockSpec((NC, RB, H), lambda i: (0, i, 0)),
            pl.BlockSpec((RB, H), lambda i: (i, 0)),
            _dinv_spec(),
            pl.BlockSpec((1, H), lambda i: (0, 0)),
            pl.BlockSpec((H, H), lambda i: (0, 0)),
        ],
        out_specs=pl.BlockSpec((RB, H), lambda i: (i, 0)),
        out_shape=jax.ShapeDtypeStruct((N, H), jnp.float32),
    )(agg, hs, dinv_rep, b_row, W)


def _mm3_body(agg_ref, hs_ref, dinv_ref, b_ref, w_ref, blin_ref, o_ref):
    a = (agg_ref[0] + agg_ref[1] + hs_ref[...]) * dinv_ref[...] + b_ref[...]
    t = jnp.maximum(a, 0.0)
    o_ref[...] = jnp.dot(t, w_ref[...],
                         preferred_element_type=jnp.float32) + blin_ref[...]


def _mm3_call(agg, hs, dinv_rep, b_row, Wlin, blin_row):
    return pl.pallas_call(
        _mm3_body,
        grid=(GRID,),
        in_specs=[
            pl.BlockSpec((NC, RB, H), lambda i: (0, i, 0)),
            pl.BlockSpec((RB, H), lambda i: (i, 0)),
            _dinv_spec(),
            pl.BlockSpec((1, H), lambda i: (0, 0)),
            pl.BlockSpec((H, OUT), lambda i: (0, 0)),
            pl.BlockSpec((1, OUT), lambda i: (0, 0)),
        ],
        out_specs=pl.BlockSpec((RB, OUT), lambda i: (i, 0)),
        out_shape=jax.ShapeDtypeStruct((N, OUT), jnp.float32),
    )(agg, hs, dinv_rep, b_row, Wlin, blin_row)


def kernel(x, edge_index, W1, b1, W2, b2, Wlin, blin):
    src = edge_index[0]
    dst = edge_index[1]
    src_p = jnp.concatenate([src, jnp.asarray(_PAD_SRC)]).reshape(NW, CPT, CHUNK)
    dst_p = jnp.concatenate([dst, jnp.asarray(_PAD_DST)]).reshape(NW, CPT, CHUNK)
    zeros1 = jnp.asarray(_ZEROS1)

    degp = _deg_kernel(dst_p, zeros1)
    dinv2d = _dinv_call(degp.reshape(NC, NROWS // 128, 128))
    dinv_rep = jnp.broadcast_to(dinv2d.reshape(NROWS)[:N, None], (N, H))

    hs1 = _mm1_call(x, W1, dinv_rep)
    agg1 = _scatter_kernel(hs1, src_p, dst_p)
    hs2 = _mm2_call(agg1, hs1, dinv_rep, b1.reshape(1, H), W2)
    agg2 = _scatter_kernel(hs2, src_p, dst_p)
    return _mm3_call(agg2, hs2, dinv_rep, b2.reshape(1, H), Wlin,
                     blin.reshape(1, OUT))


# speedup vs baseline: 33.1164x; 1.0098x over previous
import functools

import jax
import jax.numpy as jnp
import numpy as np
from jax import lax
from jax.experimental import pallas as pl
from jax.experimental.pallas import tpu as pltpu
from jax.experimental.pallas import tpu_sc as plsc

N = 10000
D = 128
H = 128
OUT = 2
E = 320000

NC = 2
NS = 16
NW = NC * NS
CHUNK = 64
CPT = 160
HCPT = CPT // 2
EPAD = NW * CPT * CHUNK
NBUF = 3
NROWS = 10112
RPT = NROWS // NS
WPT = 624
WTAIL = N - NS * WPT
RB = 2000
GRID = N // RB

_PAD = EPAD - E
_PAD_SRC = np.asarray((np.arange(_PAD) * 7919) % N, np.int32)
_PAD_DST = np.asarray(N + np.arange(_PAD) % (NROWS - N), np.int32)
_ZEROS1 = np.zeros((NROWS,), np.float32)
_ZEROS2 = np.zeros((NROWS, H), np.float32)

_mesh = plsc.VectorSubcoreMesh(core_axis_name="c", subcore_axis_name="s")


@functools.partial(
    pl.kernel,
    mesh=_mesh,
    out_type=jax.ShapeDtypeStruct((NC * NROWS,), jnp.float32),
    scratch_types=[
        pltpu.VMEM((CPT, CHUNK), jnp.int32),
        pltpu.VMEM((CHUNK,), jnp.float32),
        pltpu.VMEM_SHARED((NROWS,), jnp.float32),
        pltpu.SemaphoreType.DMA,
    ],
)
def _deg_kernel(dst_hbm, zeros_hbm, out_hbm, dst_all, ones_v, dacc, dsem):
    c = lax.axis_index("c")
    s = lax.axis_index("s")
    wid = c * NS + s
    for i in range(CHUNK // 16):
        ones_v[pl.ds(i * 16, 16)] = jnp.full((16,), 1.0, jnp.float32)
    pltpu.sync_copy(dst_hbm.at[wid], dst_all)

    @pl.when(s < NS - 1)
    def _z0():
        pltpu.sync_copy(zeros_hbm.at[pl.ds(s * 640, 640)],
                        dacc.at[pl.ds(s * 640, 640)])

    @pl.when(s == NS - 1)
    def _z1():
        pltpu.sync_copy(zeros_hbm.at[pl.ds((NS - 1) * 640, NROWS - (NS - 1) * 640)],
                        dacc.at[pl.ds((NS - 1) * 640, NROWS - (NS - 1) * 640)])

    plsc.subcore_barrier()

    def body(g, carry):
        for i in range(8):
            pltpu.async_copy(ones_v, dacc.at[dst_all.at[g * 8 + i]], dsem,
                             add=True)
        for i in range(8):
            pltpu.make_async_copy(ones_v, dacc.at[dst_all.at[g * 8 + i]],
                                  dsem).wait()
        return carry

    lax.fori_loop(0, CPT // 8, body, 0)
    plsc.subcore_barrier()

    @pl.when(s < NS - 1)
    def _w0():
        pltpu.sync_copy(dacc.at[pl.ds(s * 640, 640)],
                        out_hbm.at[pl.ds(c * NROWS + s * 640, 640)])

    @pl.when(s == NS - 1)
    def _w1():
        pltpu.sync_copy(dacc.at[pl.ds((NS - 1) * 640, NROWS - (NS - 1) * 640)],
                        out_hbm.at[pl.ds(c * NROWS + (NS - 1) * 640,
                                         NROWS - (NS - 1) * 640)])


@functools.partial(
    pl.kernel,
    mesh=_mesh,
    out_type=jax.ShapeDtypeStruct((NC, N, H), jnp.float32),
    scratch_types=[
        pltpu.VMEM((HCPT // 2, 2 * CHUNK), jnp.int32),
        pltpu.VMEM((HCPT, CHUNK), jnp.int32),
        pltpu.VMEM_SHARED((NROWS, H), jnp.float32),
    ]
    + [pltpu.VMEM((CHUNK, H), jnp.float32) for _ in range(NBUF)]
    + [pltpu.SemaphoreType.DMA for _ in range(2 * NBUF)],
)
def _scatter_kernel(hs_hbm, src_hbm, dst_hbm, out_hbm,
                    src_half, dst_half, acc, *bufs_sems):
    bufs = bufs_sems[:NBUF]
    gsem = bufs_sems[NBUF:2 * NBUF]
    ssem = bufs_sems[2 * NBUF:]
    c = lax.axis_index("c")
    s = lax.axis_index("s")
    wid = c * NS + s

    def zrow(j, carry):
        for i in range(H // 16):
            bufs[0][j, pl.ds(i * 16, 16)] = jnp.zeros((16,), jnp.float32)
        return carry

    lax.fori_loop(0, CHUNK, zrow, 0)
    nfull = RPT // CHUNK
    for k in range(nfull):
        pltpu.sync_copy(bufs[0], acc.at[pl.ds(s * RPT + k * CHUNK, CHUNK)])
    if RPT % CHUNK:
        pltpu.sync_copy(bufs[0].at[pl.ds(0, RPT % CHUNK)],
                        acc.at[pl.ds(s * RPT + nfull * CHUNK, RPT % CHUNK)])
    plsc.subcore_barrier()

    def _sidx(j):
        return src_half.at[j // 2, pl.ds((j % 2) * CHUNK, CHUNK)]

    def gather(j, b):
        pltpu.async_copy(hs_hbm.at[_sidx(j)], bufs[b], gsem[b])

    def wait_gather(j, b):
        pltpu.make_async_copy(hs_hbm.at[_sidx(j)], bufs[b], gsem[b]).wait()

    def scat(j, b):
        pltpu.async_copy(bufs[b], acc.at[dst_half.at[j]], ssem[b], add=True)

    def wait_scat(j, b):
        pltpu.make_async_copy(bufs[b], acc.at[dst_half.at[j]], ssem[b]).wait()

    for h in range(2):
        pltpu.sync_copy(src_hbm.at[wid, pl.ds(h * (HCPT // 2), HCPT // 2)],
                        src_half)
        pltpu.sync_copy(dst_hbm.at[wid, pl.ds(h * HCPT, HCPT)], dst_half)
        gather(0, 0)
        gather(1, 1)

        def body(g, carry):
            for b in range(NBUF):
                j = g * NBUF + b
                b2 = (b + 2) % NBUF
                wait_gather(j, b)
                scat(j, b)

                @pl.when(j >= 1)
                def _drain():
                    wait_scat(j - 1, b2)

                @pl.when(j + 2 < HCPT)
                def _refill():
                    gather(j + 2, b2)
            return carry

        lax.fori_loop(0, HCPT // NBUF, body, 0)
        for j in range(NBUF * (HCPT // NBUF), HCPT):
            b = j % NBUF
            wait_gather(j, b)
            scat(j, b)
            wait_scat(j - 1, (b + 2) % NBUF)
        wait_scat(HCPT - 1, (HCPT - 1) % NBUF)
    plsc.subcore_barrier()
    pltpu.sync_copy(acc.at[pl.ds(s * WPT, WPT)],
                    out_hbm.at[c, pl.ds(s * WPT, WPT)])

    @pl.when(s == NS - 1)
    def _tail():
        pltpu.sync_copy(acc.at[pl.ds(NS * WPT, WTAIL)],
                        out_hbm.at[c, pl.ds(NS * WPT, WTAIL)])


def _dinv_body(degp_ref, o_ref):
    o_ref[...] = lax.rsqrt(degp_ref[0] + degp_ref[1] + 1.0)


def _dinv_call(degp):
    return pl.pallas_call(
        _dinv_body,
        out_shape=jax.ShapeDtypeStruct((NROWS // 128, 128), jnp.float32),
    )(degp)


def _dinv_spec():
    return pl.BlockSpec((RB, H), lambda i: (i, 0))


def _mm1_body(x_ref, w_ref, dinv_ref, o_ref):
    o_ref[...] = jnp.dot(x_ref[...], w_ref[...],
                         preferred_element_type=jnp.float32) * dinv_ref[...]


def _mm1_call(x, W, dinv_rep):
    return pl.pallas_call(
        _mm1_body,
        grid=(GRID,),
        in_specs=[
            pl.BlockSpec((RB, D), lambda i: (i, 0)),
            pl.BlockSpec((D, H), lambda i: (0, 0)),
            _dinv_spec(),
        ],
        out_specs=pl.BlockSpec((RB, H), lambda i: (i, 0)),
        out_shape=jax.ShapeDtypeStruct((N, H), jnp.float32),
    )(x, W, dinv_rep)


def _mm2_body(agg_ref, hs_ref, dinv_ref, b_ref, w_ref, o_ref):
    dinv = dinv_ref[...]
    a = (agg_ref[0] + agg_ref[1] + hs_ref[...]) * dinv + b_ref[...]
    t = jnp.maximum(a, 0.0)
    o_ref[...] = jnp.dot(t, w_ref[...],
                         preferred_element_type=jnp.float32) * dinv


def _mm2_call(agg, hs, dinv_rep, b_row, W):
    return pl.pallas_call(
        _mm2_body,
        grid=(GRID,),
        in_specs=[
            pl.BlockSpec((NC, RB, H), lambda i: (0, i, 0)),
            pl.BlockSpec((RB, H), lambda i: (i, 0)),
            _dinv_spec(),
            pl.BlockSpec((1, H), lambda i: (0, 0)),
            pl.BlockSpec((H, H), lambda i: (0, 0)),
        ],
        out_specs=pl.BlockSpec((RB, H), lambda i: (i, 0)),
        out_shape=jax.ShapeDtypeStruct((N, H), jnp.float32),
    )(agg, hs, dinv_rep, b_row, W)


def _mm3_body(agg_ref, hs_ref, dinv_ref, b_ref, w_ref, blin_ref, o_ref):
    a = (agg_ref[0] + agg_ref[1] + hs_ref[...]) * dinv_ref[...] + b_ref[...]
    t = jnp.maximum(a, 0.0)
    o_ref[...] = jnp.dot(t, w_ref[...],
                         preferred_element_type=jnp.float32) + blin_ref[...]


def _mm3_call(agg, hs, dinv_rep, b_row, Wlin, blin_row):
    return pl.pallas_call(
        _mm3_body,
        grid=(GRID,),
        in_specs=[
            pl.BlockSpec((NC, RB, H), lambda i: (0, i, 0)),
            pl.BlockSpec((RB, H), lambda i: (i, 0)),
            _dinv_spec(),
            pl.BlockSpec((1, H), lambda i: (0, 0)),
            pl.BlockSpec((H, OUT), lambda i: (0, 0)),
            pl.BlockSpec((1, OUT), lambda i: (0, 0)),
        ],
        out_specs=pl.BlockSpec((RB, OUT), lambda i: (i, 0)),
        out_shape=jax.ShapeDtypeStruct((N, OUT), jnp.float32),
    )(agg, hs, dinv_rep, b_row, Wlin, blin_row)


def kernel(x, edge_index, W1, b1, W2, b2, Wlin, blin):
    src = edge_index[0]
    dst = edge_index[1]
    src_p = jnp.concatenate([src, jnp.asarray(_PAD_SRC)]).reshape(
        NW, CPT // 2, 2 * CHUNK)
    dst_p = jnp.concatenate([dst, jnp.asarray(_PAD_DST)]).reshape(NW, CPT, CHUNK)
    zeros1 = jnp.asarray(_ZEROS1)

    degp = _deg_kernel(dst_p, zeros1)
    dinv2d = _dinv_call(degp.reshape(NC, NROWS // 128, 128))
    dinv_rep = jnp.broadcast_to(dinv2d.reshape(NROWS)[:N, None], (N, H))

    hs1 = _mm1_call(x, W1, dinv_rep)
    agg1 = _scatter_kernel(hs1, src_p, dst_p)
    hs2 = _mm2_call(agg1, hs1, dinv_rep, b1.reshape(1, H), W2)
    agg2 = _scatter_kernel(hs2, src_p, dst_p)
    return _mm3_call(agg2, hs2, dinv_rep, b2.reshape(1, H), Wlin,
                     blin.reshape(1, OUT))


# speedup vs baseline: 33.1636x; 1.0014x over previous
import functools

import jax
import jax.numpy as jnp
import numpy as np
from jax import lax
from jax.experimental import pallas as pl
from jax.experimental.pallas import tpu as pltpu
from jax.experimental.pallas import tpu_sc as plsc

N = 10000
D = 128
H = 128
OUT = 2
E = 320000

NC = 2
NS = 16
NW = NC * NS
CHUNK = 64
CPT = 160
HCPT = CPT // 2
EPAD = NW * CPT * CHUNK
NBUF = 3
NROWS = 10112
RPT = NROWS // NS
WPT = 624
WTAIL = N - NS * WPT
RB = 2000
GRID = N // RB

_PAD = EPAD - E
_PAD_SRC = np.asarray((np.arange(_PAD) * 7919) % N, np.int32)
_PAD_DST = np.asarray(N + np.arange(_PAD) % (NROWS - N), np.int32)
_ZEROS1 = np.zeros((NROWS,), np.float32)

_mesh = plsc.VectorSubcoreMesh(core_axis_name="c", subcore_axis_name="s")


@functools.partial(
    pl.kernel,
    mesh=_mesh,
    out_type=jax.ShapeDtypeStruct((NC * NROWS,), jnp.float32),
    scratch_types=[
        pltpu.VMEM((CPT, CHUNK), jnp.int32),
        pltpu.VMEM((CHUNK,), jnp.float32),
        pltpu.VMEM_SHARED((NROWS,), jnp.float32),
        pltpu.SemaphoreType.DMA,
    ],
)
def _deg_kernel(dst_hbm, zeros_hbm, out_hbm, dst_all, ones_v, dacc, dsem):
    c = lax.axis_index("c")
    s = lax.axis_index("s")
    wid = c * NS + s
    for i in range(CHUNK // 16):
        ones_v[pl.ds(i * 16, 16)] = jnp.full((16,), 1.0, jnp.float32)
    pltpu.sync_copy(dst_hbm.at[wid], dst_all)

    @pl.when(s < NS - 1)
    def _z0():
        pltpu.sync_copy(zeros_hbm.at[pl.ds(s * 640, 640)],
                        dacc.at[pl.ds(s * 640, 640)])

    @pl.when(s == NS - 1)
    def _z1():
        pltpu.sync_copy(zeros_hbm.at[pl.ds((NS - 1) * 640, NROWS - (NS - 1) * 640)],
                        dacc.at[pl.ds((NS - 1) * 640, NROWS - (NS - 1) * 640)])

    plsc.subcore_barrier()

    def body(g, carry):
        for i in range(8):
            pltpu.async_copy(ones_v, dacc.at[dst_all.at[g * 8 + i]], dsem,
                             add=True)
        for i in range(8):
            pltpu.make_async_copy(ones_v, dacc.at[dst_all.at[g * 8 + i]],
                                  dsem).wait()
        return carry

    lax.fori_loop(0, CPT // 8, body, 0)
    plsc.subcore_barrier()

    @pl.when(s < NS - 1)
    def _w0():
        pltpu.sync_copy(dacc.at[pl.ds(s * 640, 640)],
                        out_hbm.at[pl.ds(c * NROWS + s * 640, 640)])

    @pl.when(s == NS - 1)
    def _w1():
        pltpu.sync_copy(dacc.at[pl.ds((NS - 1) * 640, NROWS - (NS - 1) * 640)],
                        out_hbm.at[pl.ds(c * NROWS + (NS - 1) * 640,
                                         NROWS - (NS - 1) * 640)])


@functools.partial(
    pl.kernel,
    mesh=_mesh,
    out_type=jax.ShapeDtypeStruct((NC, N, H), jnp.float32),
    scratch_types=[
        pltpu.VMEM((HCPT // 2, 2 * CHUNK), jnp.int32),
        pltpu.VMEM((HCPT, CHUNK), jnp.int32),
        pltpu.VMEM_SHARED((NROWS, H), jnp.float32),
    ]
    + [pltpu.VMEM((CHUNK, H), jnp.float32) for _ in range(NBUF)]
    + [pltpu.SemaphoreType.DMA for _ in range(2 * NBUF)],
)
def _scatter_kernel(hs_hbm, src_hbm, dst_hbm, out_hbm,
                    src_half, dst_half, acc, *bufs_sems):
    bufs = bufs_sems[:NBUF]
    gsem = bufs_sems[NBUF:2 * NBUF]
    ssem = bufs_sems[2 * NBUF:]
    c = lax.axis_index("c")
    s = lax.axis_index("s")
    wid = c * NS + s

    def zrow(j, carry):
        for i in range(H // 16):
            bufs[0][j, pl.ds(i * 16, 16)] = jnp.zeros((16,), jnp.float32)
        return carry

    lax.fori_loop(0, CHUNK, zrow, 0)
    nfull = RPT // CHUNK
    for k in range(nfull):
        pltpu.sync_copy(bufs[0], acc.at[pl.ds(s * RPT + k * CHUNK, CHUNK)])
    if RPT % CHUNK:
        pltpu.sync_copy(bufs[0].at[pl.ds(0, RPT % CHUNK)],
                        acc.at[pl.ds(s * RPT + nfull * CHUNK, RPT % CHUNK)])
    plsc.subcore_barrier()

    def _sidx(j):
        return src_half.at[j // 2, pl.ds((j % 2) * CHUNK, CHUNK)]

    def gather(j, b):
        pltpu.async_copy(hs_hbm.at[_sidx(j)], bufs[b], gsem[b])

    def wait_gather(j, b):
        pltpu.make_async_copy(hs_hbm.at[_sidx(j)], bufs[b], gsem[b]).wait()

    def scat(j, b):
        pltpu.async_copy(bufs[b], acc.at[dst_half.at[j]], ssem[b], add=True)

    def wait_scat(j, b):
        pltpu.make_async_copy(bufs[b], acc.at[dst_half.at[j]], ssem[b]).wait()

    for h in range(2):
        pltpu.sync_copy(src_hbm.at[wid, pl.ds(h * (HCPT // 2), HCPT // 2)],
                        src_half)
        pltpu.sync_copy(dst_hbm.at[wid, pl.ds(h * HCPT, HCPT)], dst_half)
        gather(0, 0)
        gather(1, 1)

        def body(g, carry):
            for b in range(NBUF):
                j = g * NBUF + b
                b2 = (b + 2) % NBUF
                wait_gather(j, b)
                scat(j, b)

                @pl.when(j >= 1)
                def _drain():
                    wait_scat(j - 1, b2)

                @pl.when(j + 2 < HCPT)
                def _refill():
                    gather(j + 2, b2)
            return carry

        lax.fori_loop(0, HCPT // NBUF, body, 0)
        for j in range(NBUF * (HCPT // NBUF), HCPT):
            b = j % NBUF
            wait_gather(j, b)
            scat(j, b)
            wait_scat(j - 1, (b + 2) % NBUF)
        wait_scat(HCPT - 1, (HCPT - 1) % NBUF)
    plsc.subcore_barrier()
    pltpu.sync_copy(acc.at[pl.ds(s * WPT, WPT)],
                    out_hbm.at[c, pl.ds(s * WPT, WPT)])

    @pl.when(s == NS - 1)
    def _tail():
        pltpu.sync_copy(acc.at[pl.ds(NS * WPT, WTAIL)],
                        out_hbm.at[c, pl.ds(NS * WPT, WTAIL)])


def _dinv_body(degp_ref, o_ref):
    o_ref[...] = lax.rsqrt(degp_ref[0] + degp_ref[1] + 1.0)


def _dinv_call(degp):
    return pl.pallas_call(
        _dinv_body,
        out_shape=jax.ShapeDtypeStruct((NROWS // 128, 128), jnp.float32),
    )(degp)


def _dinv_spec():
    return pl.BlockSpec((RB, H), lambda i: (i, 0))


def _mmraw_body(x_ref, w_ref, o_ref):
    o_ref[...] = jnp.dot(x_ref[...], w_ref[...],
                         preferred_element_type=jnp.float32)


def _mmraw_call(x, W):
    return pl.pallas_call(
        _mmraw_body,
        grid=(GRID,),
        in_specs=[
            pl.BlockSpec((RB, D), lambda i: (i, 0)),
            pl.BlockSpec((D, H), lambda i: (0, 0)),
        ],
        out_specs=pl.BlockSpec((RB, H), lambda i: (i, 0)),
        out_shape=jax.ShapeDtypeStruct((N, H), jnp.float32),
    )(x, W)


def _scale_body(h_ref, dinv_ref, o_ref):
    o_ref[...] = h_ref[...] * dinv_ref[...]


def _scale_call(h, dinv_rep):
    return pl.pallas_call(
        _scale_body,
        grid=(GRID,),
        in_specs=[
            pl.BlockSpec((RB, H), lambda i: (i, 0)),
            _dinv_spec(),
        ],
        out_specs=pl.BlockSpec((RB, H), lambda i: (i, 0)),
        out_shape=jax.ShapeDtypeStruct((N, H), jnp.float32),
    )(h, dinv_rep)


def _mm2_body(agg_ref, hs_ref, dinv_ref, b_ref, w_ref, o_ref):
    dinv = dinv_ref[...]
    a = (agg_ref[0] + agg_ref[1] + hs_ref[...]) * dinv + b_ref[...]
    t = jnp.maximum(a, 0.0)
    o_ref[...] = jnp.dot(t, w_ref[...],
                         preferred_element_type=jnp.float32) * dinv


def _mm2_call(agg, hs, dinv_rep, b_row, W):
    return pl.pallas_call(
        _mm2_body,
        grid=(GRID,),
        in_specs=[
            pl.BlockSpec((NC, RB, H), lambda i: (0, i, 0)),
            pl.BlockSpec((RB, H), lambda i: (i, 0)),
            _dinv_spec(),
            pl.BlockSpec((1, H), lambda i: (0, 0)),
            pl.BlockSpec((H, H), lambda i: (0, 0)),
        ],
        out_specs=pl.BlockSpec((RB, H), lambda i: (i, 0)),
        out_shape=jax.ShapeDtypeStruct((N, H), jnp.float32),
    )(agg, hs, dinv_rep, b_row, W)


def _mm3_body(agg_ref, hs_ref, dinv_ref, b_ref, w_ref, blin_ref, o_ref):
    a = (agg_ref[0] + agg_ref[1] + hs_ref[...]) * dinv_ref[...] + b_ref[...]
    t = jnp.maximum(a, 0.0)
    o_ref[...] = jnp.dot(t, w_ref[...],
                         preferred_element_type=jnp.float32) + blin_ref[...]


def _mm3_call(agg, hs, dinv_rep, b_row, Wlin, blin_row):
    return pl.pallas_call(
        _mm3_body,
        grid=(GRID,),
        in_specs=[
            pl.BlockSpec((NC, RB, H), lambda i: (0, i, 0)),
            pl.BlockSpec((RB, H), lambda i: (i, 0)),
            _dinv_spec(),
            pl.BlockSpec((1, H), lambda i: (0, 0)),
            pl.BlockSpec((H, OUT), lambda i: (0, 0)),
            pl.BlockSpec((1, OUT), lambda i: (0, 0)),
        ],
        out_specs=pl.BlockSpec((RB, OUT), lambda i: (i, 0)),
        out_shape=jax.ShapeDtypeStruct((N, OUT), jnp.float32),
    )(agg, hs, dinv_rep, b_row, Wlin, blin_row)


def kernel(x, edge_index, W1, b1, W2, b2, Wlin, blin):
    src = edge_index[0]
    dst = edge_index[1]
    src_p = jnp.concatenate([src, jnp.asarray(_PAD_SRC)]).reshape(
        NW, CPT // 2, 2 * CHUNK)
    dst_p = jnp.concatenate([dst, jnp.asarray(_PAD_DST)]).reshape(NW, CPT, CHUNK)
    zeros1 = jnp.asarray(_ZEROS1)

    h1 = _mmraw_call(x, W1)
    degp = _deg_kernel(dst_p, zeros1)
    dinv2d = _dinv_call(degp.reshape(NC, NROWS // 128, 128))
    dinv_rep = jnp.broadcast_to(dinv2d.reshape(NROWS)[:N, None], (N, H))

    hs1 = _scale_call(h1, dinv_rep)
    agg1 = _scatter_kernel(hs1, src_p, dst_p)
    hs2 = _mm2_call(agg1, hs1, dinv_rep, b1.reshape(1, H), W2)
    agg2 = _scatter_kernel(hs2, src_p, dst_p)
    return _mm3_call(agg2, hs2, dinv_rep, b2.reshape(1, H), Wlin,
                     blin.reshape(1, OUT))
